# baseline (speedup 1.0000x reference)
import functools
import jax
import jax.numpy as jnp
from jax import lax
from jax.experimental import pallas as pl
from jax.experimental.pallas import tpu as pltpu
from jax.experimental.pallas import tpu_sc as plsc

_L = 16


def _sc_body(col_hbm, row_hbm, out_hbm, rowbuf, rbuild, pos, sem):
    s = lax.axis_index("s")
    c = lax.axis_index("c")
    for k in range(2):
        pltpu.async_copy(
            col_hbm, pos.at[pl.ds(s * 64 + 32 * k, 32), pl.ds(0, 128)], sem
        ).wait()
    pltpu.sync_copy(row_hbm.at[pl.ds(s * 2, 2)], rowbuf)
    for r in range(2):
        for j in range(128 // _L):
            v = rowbuf[r, _L * j:_L * (j + 1)]
            for i in range(32):
                rbuild[r * 32 + i, _L * j:_L * (j + 1)] = v
    pltpu.async_copy(
        rbuild, pos.at[pl.ds(s * 64, 64), pl.ds(128, 128)], sem
    ).wait()
    plsc.subcore_barrier()
    b = c * 16 + s
    pltpu.sync_copy(pos, out_hbm.at[b])


def kernel(x, h, w, row_embed, col_embed):
    B, HW, D = x.shape
    col = jax.lax.slice(col_embed, (0, 0), (32, 128))
    row = jax.lax.slice(row_embed, (0, 0), (32, 128))
    mesh = plsc.VectorSubcoreMesh(core_axis_name="c", subcore_axis_name="s")
    k = functools.partial(
        pl.kernel,
        mesh=mesh,
        out_type=jax.ShapeDtypeStruct((B, HW, D), jnp.float32),
        scratch_types=[
            pltpu.VMEM((2, 128), jnp.float32),
            pltpu.VMEM((64, 128), jnp.float32),
            pltpu.VMEM_SHARED((HW, D), jnp.float32),
            pltpu.SemaphoreType.DMA,
        ],
    )(_sc_body)
    return k(col, row)
